# Initial kernel scaffold; baseline (speedup 1.0000x reference)
import jax
import jax.numpy as jnp
from jax.experimental import pallas as pl


def kernel(x, edge_index, W0, b0, W1, b1, W2, b2, g0, bn0, g1, bn1):
    raise NotImplementedError("write your pallas kernel here")


# speedup vs baseline: 5.1754x; 5.1754x over previous
import functools

import jax
import jax.numpy as jnp
from jax import lax
from jax.experimental import pallas as pl
from jax.experimental.pallas import tpu as pltpu
from jax.experimental.pallas import tpu_sc as plsc

EPS = 1e-5

NC = 2
NS = 16
NW = NC * NS

K = 200
ZR = 40
WT = 10


def _build_idx(dst_ref, out_ref, base, count, scale, offset):
    nfull = count // 16
    for i in range(nfull):
        v = dst_ref[pl.ds(base + i * 16, 16)]
        out_ref[pl.ds(i * 16, 16)] = v * scale + offset
    if count % 16 != 0:
        o = count - 16
        v = dst_ref[pl.ds(base + o, 16)]
        out_ref[pl.ds(o, 16)] = v * scale + offset


def _make_spmm(N, E, S, interpret=False):
    EPT = E // NW
    CHUNKS = EPT // K
    RPT = N // WT
    mesh = plsc.VectorSubcoreMesh(core_axis_name="c", subcore_axis_name="s")

    @functools.partial(
        pl.kernel,
        out_type=jax.ShapeDtypeStruct((NC * S * N, 128), jnp.float32),
        mesh=mesh,
        interpret=interpret,
        scratch_types=[
            pltpu.VMEM((EPT,), jnp.int32),
            pltpu.VMEM((EPT,), jnp.int32),
            pltpu.VMEM((K,), jnp.int32),
            pltpu.VMEM((K,), jnp.int32),
            pltpu.VMEM((K, 128), jnp.float32),
            pltpu.VMEM((ZR, 128), jnp.float32),
            pltpu.VMEM_SHARED((N, 128), jnp.float32),
            pltpu.SemaphoreType.DMA,
        ],
    )
    def spmm(table, src, dst, zeros, out,
             src_all, dst_all, gidx, sidx, rows, zbuf, acc, sem):
        cid = lax.axis_index("c")
        sid = lax.axis_index("s")
        wid = cid * NS + sid
        ebase = wid * EPT
        pltpu.sync_copy(src.at[pl.ds(ebase, EPT)], src_all)
        pltpu.sync_copy(dst.at[pl.ds(ebase, EPT)], dst_all)
        pltpu.sync_copy(zeros, zbuf)
        for s in range(S):
            plsc.subcore_barrier()

            @pl.when(sid < WT)
            def _zero():
                for z in range(RPT // ZR):
                    pltpu.sync_copy(zbuf,
                                    acc.at[pl.ds(sid * RPT + z * ZR, ZR)])

            plsc.subcore_barrier()

            def chunk(g, _):
                _build_idx(src_all, gidx, g * K, K, S, s)
                _build_idx(dst_all, sidx, g * K, K, 1, 0)
                pltpu.async_copy(table.at[gidx], rows, sem).wait()
                pltpu.sync_copy(rows, acc.at[sidx], add=True)
                return 0

            lax.fori_loop(0, CHUNKS, chunk, 0)
            plsc.subcore_barrier()

            @pl.when(sid < WT)
            def _writeout():
                obase = (cid * S + s) * N + sid * RPT
                for z in range(RPT // ZR):
                    pltpu.sync_copy(acc.at[pl.ds(sid * RPT + z * ZR, ZR)],
                                    out.at[pl.ds(obase + z * ZR, ZR)])

    return spmm


def _make_deg(N, E, interpret=False):
    EPT = E // NW
    CHUNKS = EPT // K
    RPT = N // WT
    mesh = plsc.VectorSubcoreMesh(core_axis_name="c", subcore_axis_name="s")

    @functools.partial(
        pl.kernel,
        out_type=jax.ShapeDtypeStruct((NC * N, 128), jnp.float32),
        mesh=mesh,
        interpret=interpret,
        scratch_types=[
            pltpu.VMEM((EPT,), jnp.int32),
            pltpu.VMEM((K,), jnp.int32),
            pltpu.VMEM((K, 128), jnp.float32),
            pltpu.VMEM((ZR, 128), jnp.float32),
            pltpu.VMEM_SHARED((N, 128), jnp.float32),
        ],
    )
    def deg(dst, ones, zeros, out, dst_all, sidx, obuf, zbuf, acc):
        cid = lax.axis_index("c")
        sid = lax.axis_index("s")
        wid = cid * NS + sid
        pltpu.sync_copy(dst.at[pl.ds(wid * EPT, EPT)], dst_all)
        pltpu.sync_copy(ones, obuf)
        pltpu.sync_copy(zeros, zbuf)
        plsc.subcore_barrier()

        @pl.when(sid < WT)
        def _zero():
            for z in range(RPT // ZR):
                pltpu.sync_copy(zbuf, acc.at[pl.ds(sid * RPT + z * ZR, ZR)])

        plsc.subcore_barrier()

        def chunk(g, _):
            _build_idx(dst_all, sidx, g * K, K, 1, 0)
            pltpu.sync_copy(obuf, acc.at[sidx], add=True)
            return 0

        lax.fori_loop(0, CHUNKS, chunk, 0)
        plsc.subcore_barrier()

        @pl.when(sid < WT)
        def _writeout():
            obase = cid * N + sid * RPT
            for z in range(RPT // ZR):
                pltpu.sync_copy(acc.at[pl.ds(sid * RPT + z * ZR, ZR)],
                                out.at[pl.ds(obase + z * ZR, ZR)])

    return deg


def _inv_deg(dp):
    deg = dp[0, :, 0:1] + dp[1, :, 0:1]
    return jnp.where(deg > 0.0, 1.0 / jnp.maximum(deg, 1.0), 0.0)


def _tc_layer(parts, degp, xin, wa, wx, b, g, bn, *, ln_relu, R=400,
              interpret=False):
    N, C = xin.shape
    S = C // 128
    H = wa.shape[1]

    def body(p_ref, d_ref, x_ref, wa_ref, wx_ref, b_ref, g_ref, bn_ref, o_ref):
        p = p_ref[...]
        ps = p[0] + p[1]
        inv = _inv_deg(d_ref[...])
        agg = jnp.concatenate([ps[s] for s in range(S)], axis=-1) * inv
        h = (jnp.dot(agg, wa_ref[...], preferred_element_type=jnp.float32)
             + jnp.dot(x_ref[...], wx_ref[...],
                       preferred_element_type=jnp.float32)
             + b_ref[...])
        if ln_relu:
            mu = jnp.mean(h, axis=-1, keepdims=True)
            var = jnp.mean((h - mu) ** 2, axis=-1, keepdims=True)
            h = (h - mu) * lax.rsqrt(var + EPS) * g_ref[...] + bn_ref[...]
            h = jnp.maximum(h, 0.0)
        o_ref[...] = h

    return pl.pallas_call(
        body,
        grid=(N // R,),
        in_specs=[
            pl.BlockSpec((NC, S, R, 128), lambda i: (0, 0, i, 0)),
            pl.BlockSpec((NC, R, 128), lambda i: (0, i, 0)),
            pl.BlockSpec((R, C), lambda i: (i, 0)),
            pl.BlockSpec((C, H), lambda i: (0, 0)),
            pl.BlockSpec((C, H), lambda i: (0, 0)),
            pl.BlockSpec((1, H), lambda i: (0, 0)),
            pl.BlockSpec((1, H), lambda i: (0, 0)),
            pl.BlockSpec((1, H), lambda i: (0, 0)),
        ],
        out_specs=pl.BlockSpec((R, H), lambda i: (i, 0)),
        out_shape=jax.ShapeDtypeStruct((N, H), jnp.float32),
        interpret=interpret,
    )(parts, degp, xin, wa, wx, b, g, bn)


def _tc_lin2(xin, wa, wx, b, *, R=400, interpret=False):
    N, C = xin.shape
    H = wa.shape[1]

    def body(x_ref, wa_ref, wx_ref, b_ref, ya_ref, yx_ref):
        xb = x_ref[...]
        ya_ref[...] = jnp.dot(xb, wa_ref[...],
                              preferred_element_type=jnp.float32)
        yx_ref[...] = jnp.dot(xb, wx_ref[...],
                              preferred_element_type=jnp.float32) + b_ref[...]

    return pl.pallas_call(
        body,
        grid=(N // R,),
        in_specs=[
            pl.BlockSpec((R, C), lambda i: (i, 0)),
            pl.BlockSpec((C, H), lambda i: (0, 0)),
            pl.BlockSpec((C, H), lambda i: (0, 0)),
            pl.BlockSpec((1, H), lambda i: (0, 0)),
        ],
        out_specs=[
            pl.BlockSpec((R, H), lambda i: (i, 0)),
            pl.BlockSpec((R, H), lambda i: (i, 0)),
        ],
        out_shape=[
            jax.ShapeDtypeStruct((N, H), jnp.float32),
            jax.ShapeDtypeStruct((N, H), jnp.float32),
        ],
        interpret=interpret,
    )(xin, wa, wx, b)


def _tc_final(parts, degp, yx, *, R=400, interpret=False):
    N, H = yx.shape
    S = H // 128

    def body(p_ref, d_ref, y_ref, o_ref):
        p = p_ref[...]
        ps = p[0] + p[1]
        inv = _inv_deg(d_ref[...])
        agg = jnp.concatenate([ps[s] for s in range(S)], axis=-1) * inv
        o_ref[...] = agg + y_ref[...]

    return pl.pallas_call(
        body,
        grid=(N // R,),
        in_specs=[
            pl.BlockSpec((NC, S, R, 128), lambda i: (0, 0, i, 0)),
            pl.BlockSpec((NC, R, 128), lambda i: (0, i, 0)),
            pl.BlockSpec((R, H), lambda i: (i, 0)),
        ],
        out_specs=pl.BlockSpec((R, H), lambda i: (i, 0)),
        out_shape=jax.ShapeDtypeStruct((N, H), jnp.float32),
        interpret=interpret,
    )(parts, degp, yx)


def kernel(x, edge_index, W0, b0, W1, b1, W2, b2, g0, bn0, g1, bn1):
    N, C0 = x.shape
    E = edge_index.shape[1]
    H = W0.shape[0]
    OUT = W2.shape[0]
    src = edge_index[0].astype(jnp.int32)
    dst = edge_index[1].astype(jnp.int32)

    Wt0 = W0.T
    Wt1 = W1.T
    Wt2 = W2.T
    wa0, wx0 = Wt0[:C0], Wt0[C0:]
    wa1, wx1 = Wt1[:H], Wt1[H:]
    wa2, wx2 = Wt2[:H], Wt2[H:]
    b0r, g0r, bn0r = b0.reshape(1, -1), g0.reshape(1, -1), bn0.reshape(1, -1)
    b1r, g1r, bn1r = b1.reshape(1, -1), g1.reshape(1, -1), bn1.reshape(1, -1)
    b2r = b2.reshape(1, -1)

    z128 = jnp.zeros((ZR, 128), jnp.float32)
    o128 = jnp.ones((K, 128), jnp.float32)

    S0 = C0 // 128
    SH = H // 128
    SO = OUT // 128

    degp = _make_deg(N, E)(dst, o128, z128).reshape(NC, N, 128)

    p0 = _make_spmm(N, E, S0)(x.reshape(-1, 128), src, dst, z128)
    h0 = _tc_layer(p0.reshape(NC, S0, N, 128), degp, x,
                   wa0, wx0, b0r, g0r, bn0r, ln_relu=True)

    p1 = _make_spmm(N, E, SH)(h0.reshape(-1, 128), src, dst, z128)
    h1 = _tc_layer(p1.reshape(NC, SH, N, 128), degp, h0,
                   wa1, wx1, b1r, g1r, bn1r, ln_relu=True)

    ya, yx = _tc_lin2(h1, wa2, wx2, b2r)
    p2 = _make_spmm(N, E, SO)(ya.reshape(-1, 128), src, dst, z128)
    out = _tc_final(p2.reshape(NC, SO, N, 128), degp, yx)
    return out
